# Initial kernel scaffold; baseline (speedup 1.0000x reference)
import jax
import jax.numpy as jnp
from jax.experimental import pallas as pl


def kernel(input_tensor, position_vectors):
    raise NotImplementedError("write your pallas kernel here")


# speedup vs baseline: 1.5465x; 1.5465x over previous
import jax
import jax.numpy as jnp
from jax.experimental import pallas as pl

_S_BLK = 512


def _bind_kernel(x_ref, p_ref, o_ref):
    o_ref[...] = x_ref[...] * p_ref[...]


def kernel(input_tensor, position_vectors):
    bsz, seq_len, d = input_tensor.shape
    pos = position_vectors[:seq_len, :d]
    grid = (seq_len // _S_BLK, bsz)
    return pl.pallas_call(
        _bind_kernel,
        grid=grid,
        in_specs=[
            pl.BlockSpec((1, _S_BLK, d), lambda s, b: (b, s, 0)),
            pl.BlockSpec((_S_BLK, d), lambda s, b: (s, 0)),
        ],
        out_specs=pl.BlockSpec((1, _S_BLK, d), lambda s, b: (b, s, 0)),
        out_shape=jax.ShapeDtypeStruct((bsz, seq_len, d), input_tensor.dtype),
    )(input_tensor, pos)


# speedup vs baseline: 1.5947x; 1.0312x over previous
import jax
import jax.numpy as jnp
from jax.experimental import pallas as pl

_S_BLK = 1024


def _bind_kernel(x_ref, p_ref, o_ref):
    o_ref[...] = x_ref[...] * p_ref[...]


def kernel(input_tensor, position_vectors):
    bsz, seq_len, d = input_tensor.shape
    pos = position_vectors[:seq_len, :d]
    grid = (seq_len // _S_BLK, bsz)
    return pl.pallas_call(
        _bind_kernel,
        grid=grid,
        in_specs=[
            pl.BlockSpec((1, _S_BLK, d), lambda s, b: (b, s, 0)),
            pl.BlockSpec((_S_BLK, d), lambda s, b: (s, 0)),
        ],
        out_specs=pl.BlockSpec((1, _S_BLK, d), lambda s, b: (b, s, 0)),
        out_shape=jax.ShapeDtypeStruct((bsz, seq_len, d), input_tensor.dtype),
    )(input_tensor, pos)
